# Initial kernel scaffold; baseline (speedup 1.0000x reference)
import jax
import jax.numpy as jnp
from jax.experimental import pallas as pl


def kernel(prop_type, ent_ix, rel_ix, quals_ix, ent_embs, rel_embs, edge_index, edge_type, quals, w_in1, w_out1, w_loop1, w_rel1, w_in2, w_out2, w_loop2, w_rel2, loop_rel1, loop_rel2, b1, b2):
    raise NotImplementedError("write your pallas kernel here")


# speedup vs baseline: 2.6496x; 2.6496x over previous
import functools

import jax
import jax.numpy as jnp
from jax import lax
from jax.experimental import pallas as pl
from jax.experimental.pallas import tpu as pltpu
from jax.experimental.pallas import tpu_sc as plsc

N_ENT = 10000
N_EDGE = 160000
N_REL = 400
D = 128
NQ = 40000
ALPHA = 0.8
HALF = N_EDGE // 2

NC = 2
NS = 16
K = 128
NROWS = 10240
ROWS_PER_TILE = NROWS // NS
ZROWS = 160
ECHUNKS = HALF // K
NQP = 40960
QCHUNKS = NQP // K
DUMMY = N_ENT

_mesh = plsc.VectorSubcoreMesh(core_axis_name="c", subcore_axis_name="s")


@functools.partial(
    pl.kernel,
    out_type=jax.ShapeDtypeStruct((NC, NROWS, D), jnp.float32),
    mesh=_mesh,
    scratch_types=[
        pltpu.VMEM_SHARED((NROWS, D), jnp.float32),
        pltpu.VMEM((K,), jnp.int32),
        pltpu.VMEM((K, D), jnp.float32),
    ],
)
def _cnt_kernel(dst_hbm, z128_hbm, ones_hbm, c_hbm, cnt, didx, ones):
    c = lax.axis_index("c")
    s = lax.axis_index("s")
    r0 = s * ROWS_PER_TILE
    for j in range(ROWS_PER_TILE // ZROWS):
        pltpu.sync_copy(z128_hbm, cnt.at[pl.ds(r0 + j * ZROWS, ZROWS)])
    pltpu.sync_copy(ones_hbm, ones)
    plsc.subcore_barrier()

    @pl.loop(s, ECHUNKS, step=NS)
    def _(j):
        pltpu.sync_copy(dst_hbm.at[pl.ds(c * HALF + j * K, K)], didx)
        pltpu.sync_copy(ones, cnt.at[didx], add=True)

    plsc.subcore_barrier()
    pltpu.sync_copy(cnt.at[pl.ds(r0, ROWS_PER_TILE)],
                    c_hbm.at[c, pl.ds(r0, ROWS_PER_TILE)])


def _make_edge_kernel():
    @functools.partial(
        pl.kernel,
        out_type=jax.ShapeDtypeStruct((NC, NROWS, D), jnp.float32),
        mesh=_mesh,
        scratch_types=[
            pltpu.VMEM_SHARED((NROWS, D), jnp.float32),
            pltpu.VMEM((K, D), jnp.float32),
            pltpu.VMEM((K, D), jnp.float32),
            pltpu.VMEM((K,), jnp.int32),
            pltpu.VMEM((K,), jnp.int32),
            pltpu.VMEM((K,), jnp.int32),
        ],
    )
    def edge_kernel(x_hbm, rneg_hbm, rq_hbm, src_hbm, dst_hbm, et_hbm,
                    qrel_hbm, qent_hbm, qd2_hbm, z128_hbm,
                    a_hbm, acc, xbuf, rbuf, sidx, tidx, didx):
        c = lax.axis_index("c")
        s = lax.axis_index("s")

        r0 = s * ROWS_PER_TILE
        for j in range(ROWS_PER_TILE // ZROWS):
            pltpu.sync_copy(z128_hbm, acc.at[pl.ds(r0 + j * ZROWS, ZROWS)])
        plsc.subcore_barrier()

        @pl.loop(s, ECHUNKS, step=NS)
        def _(j):
            e0 = c * HALF + j * K
            pltpu.sync_copy(src_hbm.at[pl.ds(e0, K)], sidx)
            pltpu.sync_copy(et_hbm.at[pl.ds(e0, K)], tidx)
            pltpu.sync_copy(dst_hbm.at[pl.ds(e0, K)], didx)
            pltpu.sync_copy(x_hbm.at[sidx], xbuf)
            pltpu.sync_copy(rneg_hbm.at[tidx], rbuf)
            pltpu.sync_copy(xbuf, acc.at[didx], add=True)
            pltpu.sync_copy(rbuf, acc.at[didx], add=True)

        @pl.loop(s, QCHUNKS, step=NS)
        def _(j):
            q0 = j * K
            pltpu.sync_copy(qrel_hbm.at[pl.ds(q0, K)], tidx)
            pltpu.sync_copy(qent_hbm.at[pl.ds(q0, K)], sidx)
            pltpu.sync_copy(qd2_hbm.at[c, pl.ds(q0, K)], didx)
            pltpu.sync_copy(rq_hbm.at[tidx], rbuf)
            pltpu.sync_copy(x_hbm.at[sidx], xbuf)

            @pl.loop(0, K)
            def _(i):
                @pl.loop(0, D, step=16)
                def _(k):
                    xbuf[i, pl.ds(k, 16)] = (
                        xbuf[i, pl.ds(k, 16)] * rbuf[i, pl.ds(k, 16)])

            pltpu.sync_copy(xbuf, acc.at[didx], add=True)

        plsc.subcore_barrier()
        pltpu.sync_copy(acc.at[pl.ds(r0, ROWS_PER_TILE)],
                        a_hbm.at[c, pl.ds(r0, ROWS_PER_TILE)])

    return edge_kernel


_edge_kernel = _make_edge_kernel()


NGX = 7168
GCHUNKS = NGX // K


@functools.partial(
    pl.kernel,
    out_type=(jax.ShapeDtypeStruct((NGX, D), jnp.float32),
              jax.ShapeDtypeStruct((NGX, D), jnp.float32)),
    mesh=_mesh,
    scratch_types=[
        pltpu.VMEM((K,), jnp.int32),
        pltpu.VMEM((K, D), jnp.float32),
    ],
)
def _gather_kernel(x_hbm, r_hbm, ix_hbm, ir_hbm, ox_hbm, or_hbm, ibuf, gbuf):
    c = lax.axis_index("c")
    s = lax.axis_index("s")
    w = s * NC + c

    @pl.loop(w, GCHUNKS, step=NC * NS)
    def _(j):
        pltpu.sync_copy(ix_hbm.at[pl.ds(j * K, K)], ibuf)
        pltpu.sync_copy(x_hbm.at[ibuf], gbuf)
        pltpu.sync_copy(gbuf, ox_hbm.at[pl.ds(j * K, K)])

    @pl.loop(w, GCHUNKS, step=NC * NS)
    def _(j):
        pltpu.sync_copy(ir_hbm.at[pl.ds(j * K, K)], ibuf)
        pltpu.sync_copy(r_hbm.at[ibuf], gbuf)
        pltpu.sync_copy(gbuf, or_hbm.at[pl.ds(j * K, K)])


def _tc_prep_body(r_ref, wr1_ref, wr2_ref,
                  rneg1_ref, rq1_ref, r2_ref, rneg2_ref, rq2_ref, rfin_ref):
    r = r_ref[...]
    rneg1_ref[...] = (-ALPHA) * r
    rq1_ref[...] = (ALPHA - 1.0) * r
    r2 = jnp.dot(r, wr1_ref[...], preferred_element_type=jnp.float32)
    r2_ref[...] = r2
    rneg2_ref[...] = (-ALPHA) * r2
    rq2_ref[...] = (ALPHA - 1.0) * r2
    rfin_ref[...] = jnp.dot(r2, wr2_ref[...], preferred_element_type=jnp.float32)


def _tc_prep(r, wr1, wr2):
    sds = jax.ShapeDtypeStruct((N_REL, D), jnp.float32)
    return pl.pallas_call(
        _tc_prep_body,
        out_shape=(sds,) * 6,
    )(r, wr1, wr2)


def _tc_dense_body(ain_ref, aout_ref, cin_ref, cout_ref, x_ref,
                   win_ref, wout_ref, wloop_ref, lr_ref, b_ref, out_ref):
    norm_in = 1.0 / jnp.maximum(cin_ref[:, 0:1], 1.0)
    norm_out = 1.0 / jnp.maximum(cout_ref[:, 0:1], 1.0)
    x = x_ref[...]
    t = jnp.dot(ain_ref[...] * norm_in, win_ref[...],
                preferred_element_type=jnp.float32)
    t += jnp.dot(aout_ref[...] * norm_out, wout_ref[...],
                 preferred_element_type=jnp.float32)
    t += jnp.dot(x - lr_ref[...], wloop_ref[...],
                 preferred_element_type=jnp.float32)
    out_ref[...] = jnp.tanh(t * (1.0 / 3.0) + b_ref[...])


def _tc_dense(ain, aout, cin, cout, x, w_in, w_out, w_loop, loop_rel, b):
    return pl.pallas_call(
        _tc_dense_body,
        out_shape=jax.ShapeDtypeStruct((N_ENT, D), jnp.float32),
    )(ain, aout, cin, cout, x, w_in, w_out, w_loop, loop_rel, b.reshape(1, D))


def kernel(prop_type, ent_ix, rel_ix, quals_ix, ent_embs, rel_embs,
           edge_index, edge_type, quals,
           w_in1, w_out1, w_loop1, w_rel1, w_in2, w_out2, w_loop2, w_rel2,
           loop_rel1, loop_rel2, b1, b2):
    i32 = jnp.int32
    src = edge_index[0].astype(i32)
    dst = edge_index[1].astype(i32)
    et = edge_type.astype(i32)

    eid = quals[2].astype(i32)
    qd = dst[eid]
    pad = NQP - NQ
    qd_in = jnp.concatenate(
        [jnp.where(eid < HALF, qd, DUMMY), jnp.full((pad,), DUMMY, i32)])
    qd_out = jnp.concatenate(
        [jnp.where(eid >= HALF, qd, DUMMY), jnp.full((pad,), DUMMY, i32)])
    qd2 = jnp.stack([qd_in, qd_out])
    qrel = jnp.concatenate([quals[0].astype(i32), jnp.zeros((pad,), i32)])
    qent = jnp.concatenate([quals[1].astype(i32), jnp.zeros((pad,), i32)])

    z128 = jnp.zeros((ZROWS, D), jnp.float32)
    ones128 = jnp.ones((K, D), jnp.float32)

    rneg1, rq1, r2, rneg2, rq2, rfin = _tc_prep(rel_embs, w_rel1, w_rel2)

    c1 = _cnt_kernel(dst, z128, ones128)

    a1 = _edge_kernel(ent_embs, rneg1, rq1, src, dst, et,
                      qrel, qent, qd2, z128)
    x2 = _tc_dense(a1[0, :N_ENT], a1[1, :N_ENT], c1[0, :N_ENT], c1[1, :N_ENT],
                   ent_embs, w_in1, w_out1, w_loop1, loop_rel1, b1)

    a2 = _edge_kernel(x2, rneg2, rq2, src, dst, et,
                      qrel, qent, qd2, z128)
    x3 = _tc_dense(a2[0, :N_ENT], a2[1, :N_ENT], c1[0, :N_ENT], c1[1, :N_ENT],
                   x2, w_in2, w_out2, w_loop2, loop_rel2, b2)

    idx_x = jnp.concatenate(
        [ent_ix.astype(i32), quals_ix[:, 1::2].reshape(-1).astype(i32)])
    idx_r = jnp.concatenate(
        [rel_ix.astype(i32), quals_ix[:, 0::2].reshape(-1).astype(i32)])
    gx, gr = _gather_kernel(x3, rfin, idx_x, idx_r)

    B = ent_ix.shape[0]
    sub_emb = gx[:B]
    qual_obj_emb = gx[B:].reshape(B, -1, D)
    rel_emb = gr[:B]
    qual_rel_emb = gr[B:].reshape(B, -1, D)
    return (sub_emb, rel_emb, qual_obj_emb, qual_rel_emb, x3, rfin)


# speedup vs baseline: 3.5088x; 1.3243x over previous
import functools

import jax
import jax.numpy as jnp
from jax import lax
from jax.experimental import pallas as pl
from jax.experimental.pallas import tpu as pltpu
from jax.experimental.pallas import tpu_sc as plsc

N_ENT = 10000
N_EDGE = 160000
N_REL = 400
D = 128
NQ = 40000
ALPHA = 0.8
HALF = N_EDGE // 2

NC = 2
NS = 16
K = 128
NROWS = 10240
ROWS_PER_TILE = NROWS // NS
ZROWS = 160
ECHUNKS = HALF // K
NQP = 40960
QCHUNKS = NQP // K
DUMMY = N_ENT

_mesh = plsc.VectorSubcoreMesh(core_axis_name="c", subcore_axis_name="s")


@functools.partial(
    pl.kernel,
    out_type=jax.ShapeDtypeStruct((NC, NROWS, D), jnp.float32),
    mesh=_mesh,
    scratch_types=[
        pltpu.VMEM_SHARED((NROWS, D), jnp.float32),
        pltpu.VMEM((K,), jnp.int32),
        pltpu.VMEM((K, D), jnp.float32),
    ],
)
def _cnt_kernel(dst_hbm, z128_hbm, ones_hbm, c_hbm, cnt, didx, ones):
    c = lax.axis_index("c")
    s = lax.axis_index("s")
    r0 = s * ROWS_PER_TILE
    for j in range(ROWS_PER_TILE // ZROWS):
        pltpu.sync_copy(z128_hbm, cnt.at[pl.ds(r0 + j * ZROWS, ZROWS)])
    pltpu.sync_copy(ones_hbm, ones)
    plsc.subcore_barrier()

    @pl.loop(s, ECHUNKS, step=NS)
    def _(j):
        pltpu.sync_copy(dst_hbm.at[pl.ds(c * HALF + j * K, K)], didx)
        pltpu.sync_copy(ones, cnt.at[didx], add=True)

    plsc.subcore_barrier()
    pltpu.sync_copy(cnt.at[pl.ds(r0, ROWS_PER_TILE)],
                    c_hbm.at[c, pl.ds(r0, ROWS_PER_TILE)])


@functools.partial(
    pl.kernel,
    out_type=jax.ShapeDtypeStruct((NC, NROWS, D), jnp.float32),
    mesh=_mesh,
    scratch_types=[
        pltpu.VMEM_SHARED((NROWS, D), jnp.float32),
        pltpu.VMEM((K, D), jnp.float32),
        pltpu.VMEM((K, D), jnp.float32),
        pltpu.VMEM((3, K), jnp.int32),
        pltpu.SemaphoreType.DMA,
        pltpu.SemaphoreType.DMA,
    ],
)
def _edge_kernel(x_hbm, rneg_hbm, rq_hbm, eidx_hbm, qidx_hbm, z128_hbm,
                 a_hbm, acc, xbuf, rbuf, ibuf, semx, semr):
    c = lax.axis_index("c")
    s = lax.axis_index("s")

    r0 = s * ROWS_PER_TILE
    for j in range(ROWS_PER_TILE // ZROWS):
        pltpu.sync_copy(z128_hbm, acc.at[pl.ds(r0 + j * ZROWS, ZROWS)])
    plsc.subcore_barrier()

    @pl.loop(s, ECHUNKS, step=NS)
    def _(j):
        pltpu.sync_copy(eidx_hbm.at[:, pl.ds(c * HALF + j * K, K)], ibuf)
        gx = pltpu.async_copy(x_hbm.at[ibuf.at[0]], xbuf, semx)
        gr = pltpu.async_copy(rneg_hbm.at[ibuf.at[1]], rbuf, semr)
        gx.wait()
        sx = pltpu.async_copy(xbuf, acc.at[ibuf.at[2]], semx, add=True)
        gr.wait()
        sr = pltpu.async_copy(rbuf, acc.at[ibuf.at[2]], semr, add=True)
        sx.wait()
        sr.wait()

    @pl.loop(s, QCHUNKS, step=NS)
    def _(j):
        pltpu.sync_copy(qidx_hbm.at[c, :, pl.ds(j * K, K)], ibuf)
        gr = pltpu.async_copy(rq_hbm.at[ibuf.at[0]], rbuf, semr)
        gx = pltpu.async_copy(x_hbm.at[ibuf.at[1]], xbuf, semx)
        gr.wait()
        gx.wait()

        @pl.loop(0, K)
        def _(i):
            @pl.loop(0, D, step=16)
            def _(k):
                xbuf[i, pl.ds(k, 16)] = (
                    xbuf[i, pl.ds(k, 16)] * rbuf[i, pl.ds(k, 16)])

        pltpu.sync_copy(xbuf, acc.at[ibuf.at[2]], add=True)

    plsc.subcore_barrier()
    pltpu.sync_copy(acc.at[pl.ds(r0, ROWS_PER_TILE)],
                    a_hbm.at[c, pl.ds(r0, ROWS_PER_TILE)])


NGX = 7168
GCHUNKS = NGX // K


@functools.partial(
    pl.kernel,
    out_type=(jax.ShapeDtypeStruct((NGX, D), jnp.float32),
              jax.ShapeDtypeStruct((NGX, D), jnp.float32)),
    mesh=_mesh,
    scratch_types=[
        pltpu.VMEM((K,), jnp.int32),
        pltpu.VMEM((K, D), jnp.float32),
    ],
)
def _gather_kernel(x_hbm, r_hbm, ix_hbm, ir_hbm, ox_hbm, or_hbm, ibuf, gbuf):
    c = lax.axis_index("c")
    s = lax.axis_index("s")
    w = s * NC + c

    @pl.loop(w, GCHUNKS, step=NC * NS)
    def _(j):
        pltpu.sync_copy(ix_hbm.at[pl.ds(j * K, K)], ibuf)
        pltpu.sync_copy(x_hbm.at[ibuf], gbuf)
        pltpu.sync_copy(gbuf, ox_hbm.at[pl.ds(j * K, K)])

    @pl.loop(w, GCHUNKS, step=NC * NS)
    def _(j):
        pltpu.sync_copy(ir_hbm.at[pl.ds(j * K, K)], ibuf)
        pltpu.sync_copy(r_hbm.at[ibuf], gbuf)
        pltpu.sync_copy(gbuf, or_hbm.at[pl.ds(j * K, K)])


def _tc_prep_body(r_ref, wr1_ref, wr2_ref,
                  rneg1_ref, rq1_ref, r2_ref, rneg2_ref, rq2_ref, rfin_ref):
    r = r_ref[...]
    rneg1_ref[...] = (-ALPHA) * r
    rq1_ref[...] = (ALPHA - 1.0) * r
    r2 = jnp.dot(r, wr1_ref[...], preferred_element_type=jnp.float32)
    r2_ref[...] = r2
    rneg2_ref[...] = (-ALPHA) * r2
    rq2_ref[...] = (ALPHA - 1.0) * r2
    rfin_ref[...] = jnp.dot(r2, wr2_ref[...], preferred_element_type=jnp.float32)


def _tc_prep(r, wr1, wr2):
    sds = jax.ShapeDtypeStruct((N_REL, D), jnp.float32)
    return pl.pallas_call(
        _tc_prep_body,
        out_shape=(sds,) * 6,
    )(r, wr1, wr2)


def _tc_dense_body(a_ref, c_ref, x_ref,
                   win_ref, wout_ref, wloop_ref, lr_ref, b_ref, out_ref):
    norm_in = 1.0 / jnp.maximum(c_ref[0, :N_ENT, 0:1], 1.0)
    norm_out = 1.0 / jnp.maximum(c_ref[1, :N_ENT, 0:1], 1.0)
    x = x_ref[...]
    t = jnp.dot(a_ref[0, :N_ENT, :] * norm_in, win_ref[...],
                preferred_element_type=jnp.float32)
    t += jnp.dot(a_ref[1, :N_ENT, :] * norm_out, wout_ref[...],
                 preferred_element_type=jnp.float32)
    t += jnp.dot(x - lr_ref[...], wloop_ref[...],
                 preferred_element_type=jnp.float32)
    out_ref[...] = jnp.tanh(t * (1.0 / 3.0) + b_ref[...])


def _tc_dense(a, cnt, x, w_in, w_out, w_loop, loop_rel, b):
    return pl.pallas_call(
        _tc_dense_body,
        out_shape=jax.ShapeDtypeStruct((N_ENT, D), jnp.float32),
    )(a, cnt, x, w_in, w_out, w_loop, loop_rel, b.reshape(1, D))


def kernel(prop_type, ent_ix, rel_ix, quals_ix, ent_embs, rel_embs,
           edge_index, edge_type, quals,
           w_in1, w_out1, w_loop1, w_rel1, w_in2, w_out2, w_loop2, w_rel2,
           loop_rel1, loop_rel2, b1, b2):
    i32 = jnp.int32
    src = edge_index[0].astype(i32)
    dst = edge_index[1].astype(i32)
    et = edge_type.astype(i32)
    eidx = jnp.stack([src, et, dst])

    eid = quals[2].astype(i32)
    qd = dst[eid]
    pad = NQP - NQ
    qd_in = jnp.concatenate(
        [jnp.where(eid < HALF, qd, DUMMY), jnp.full((pad,), DUMMY, i32)])
    qd_out = jnp.concatenate(
        [jnp.where(eid >= HALF, qd, DUMMY), jnp.full((pad,), DUMMY, i32)])
    qrel = jnp.concatenate([quals[0].astype(i32), jnp.zeros((pad,), i32)])
    qent = jnp.concatenate([quals[1].astype(i32), jnp.zeros((pad,), i32)])
    qidx = jnp.stack([jnp.stack([qrel, qent, qd_in]),
                      jnp.stack([qrel, qent, qd_out])])

    z128 = jnp.zeros((ZROWS, D), jnp.float32)
    ones128 = jnp.ones((K, D), jnp.float32)

    rneg1, rq1, r2, rneg2, rq2, rfin = _tc_prep(rel_embs, w_rel1, w_rel2)

    c1 = _cnt_kernel(dst, z128, ones128)

    a1 = _edge_kernel(ent_embs, rneg1, rq1, eidx, qidx, z128)
    x2 = _tc_dense(a1, c1, ent_embs, w_in1, w_out1, w_loop1, loop_rel1, b1)

    a2 = _edge_kernel(x2, rneg2, rq2, eidx, qidx, z128)
    x3 = _tc_dense(a2, c1, x2, w_in2, w_out2, w_loop2, loop_rel2, b2)

    idx_x = jnp.concatenate(
        [ent_ix.astype(i32), quals_ix[:, 1::2].reshape(-1).astype(i32)])
    idx_r = jnp.concatenate(
        [rel_ix.astype(i32), quals_ix[:, 0::2].reshape(-1).astype(i32)])
    gx, gr = _gather_kernel(x3, rfin, idx_x, idx_r)

    B = ent_ix.shape[0]
    sub_emb = gx[:B]
    qual_obj_emb = gx[B:].reshape(B, -1, D)
    rel_emb = gr[:B]
    qual_rel_emb = gr[B:].reshape(B, -1, D)
    return (sub_emb, rel_emb, qual_obj_emb, qual_rel_emb, x3, rfin)
